# baseline (speedup 1.0000x reference)
import functools

import jax
import jax.numpy as jnp
from jax import lax
from jax.experimental import pallas as pl
from jax.experimental.pallas import tpu as pltpu
from jax.experimental.pallas import tpu_sc as plsc

EMB = 64
BATCH = 16384
N_MONTH = 12
N_HOUR = 24


def _build_call():
    info = plsc.get_sparse_core_info()
    nc, ns = info.num_cores, info.num_subcores
    nw = nc * ns
    bpw = BATCH // nw
    nchunk = bpw // 64

    mesh = plsc.VectorSubcoreMesh(core_axis_name="c", subcore_axis_name="s")

    @functools.partial(
        pl.kernel,
        mesh=mesh,
        out_type=jax.ShapeDtypeStruct((BATCH, 2 * EMB), jnp.float32),
        scratch_types=[
            pltpu.VMEM((bpw,), jnp.int32),
            pltpu.VMEM((bpw,), jnp.int32),
            pltpu.VMEM((nchunk, 64), jnp.int32),
            pltpu.VMEM((bpw, 2 * EMB), jnp.float32),
            pltpu.VMEM((N_HOUR, 2 * EMB), jnp.float32),
            pltpu.VMEM((N_HOUR, EMB), jnp.float32),
            pltpu.VMEM_SHARED((N_MONTH * N_HOUR, 2 * EMB), jnp.float32),
            pltpu.SemaphoreType.DMA,
            pltpu.SemaphoreType.DMA,
            pltpu.SemaphoreType.DMA,
            pltpu.SemaphoreType.DMA,
        ],
    )
    def fused(mon_hbm, hr_hbm, months_hbm, hours_hbm, out_hbm,
              mon_v, hr_v, idx_v, rows_v, buf_v, hbuf_v, tbl2_s,
              sem_i, sem_b, sem_g, sem_w):
        cid = lax.axis_index("c")
        sid = lax.axis_index("s")
        wid = sid * nc + cid
        base = wid * bpw

        ld_m = pltpu.async_copy(months_hbm.at[pl.ds(base, bpw)], mon_v, sem_i)
        ld_h = pltpu.async_copy(hours_hbm.at[pl.ds(base, bpw)], hr_v, sem_i)

        @pl.when(sid < N_MONTH)
        def _():
            stage = [
                pltpu.async_copy(mon_hbm.at[sid],
                                 buf_v.at[0, pl.ds(0, EMB)], sem_b),
                pltpu.async_copy(hr_hbm, hbuf_v, sem_b),
            ]
            for c in stage:
                c.wait()
            for c4 in range(EMB // 16):
                v0 = buf_v[0, pl.ds(c4 * 16, 16)]

                def rep(h, carry):
                    buf_v[h, pl.ds(c4 * 16, 16)] = carry
                    buf_v[h, pl.ds(EMB + c4 * 16, 16)] = hbuf_v[h, pl.ds(c4 * 16, 16)]
                    return carry

                def rep0(h, carry):
                    buf_v[h, pl.ds(EMB + c4 * 16, 16)] = hbuf_v[h, pl.ds(c4 * 16, 16)]
                    return carry

                lax.fori_loop(1, N_HOUR, rep, v0)
                lax.fori_loop(0, 1, rep0, 0)
            pltpu.async_copy(
                buf_v, tbl2_s.at[pl.ds(sid * N_HOUR, N_HOUR)], sem_b
            ).wait()

        ld_m.wait()
        ld_h.wait()

        def body(j, carry):
            m = mon_v[pl.ds(j * 16, 16)]
            h = hr_v[pl.ds(j * 16, 16)]
            r = lax.shift_right_logical(j, 2)
            cb = (j & 3) * 16
            idx_v[r, pl.ds(cb, 16)] = m * N_HOUR + h
            return carry

        lax.fori_loop(0, bpw // 16, body, 0)

        plsc.subcore_barrier()

        gathers = [
            pltpu.async_copy(tbl2_s.at[idx_v.at[k]],
                             rows_v.at[pl.ds(k * 64, 64)], sem_g)
            for k in range(nchunk)
        ]
        writes = []
        for k in range(nchunk):
            gathers[k].wait()
            writes.append(
                pltpu.async_copy(rows_v.at[pl.ds(k * 64, 64)],
                                 out_hbm.at[pl.ds(base + k * 64, 64)],
                                 sem_w))
        for w in writes:
            w.wait()

    return fused


def kernel(months, hours, month_table, hour_table):
    return _build_call()(month_table, hour_table,
                         months.astype(jnp.int32),
                         hours.astype(jnp.int32))


# speedup vs baseline: 1.0220x; 1.0220x over previous
import functools

import jax
import jax.numpy as jnp
from jax import lax
from jax.experimental import pallas as pl
from jax.experimental.pallas import tpu as pltpu
from jax.experimental.pallas import tpu_sc as plsc

EMB = 64
BATCH = 16384
N_MONTH = 12
N_HOUR = 24


def _build_call():
    info = plsc.get_sparse_core_info()
    nc, ns = info.num_cores, info.num_subcores
    nw = nc * ns
    bpw = BATCH // nw
    nchunk = bpw // 64

    mesh = plsc.VectorSubcoreMesh(core_axis_name="c", subcore_axis_name="s")

    @functools.partial(
        pl.kernel,
        mesh=mesh,
        out_type=jax.ShapeDtypeStruct((BATCH, 2 * EMB), jnp.float32),
        scratch_types=[
            pltpu.VMEM((bpw,), jnp.int32),
            pltpu.VMEM((bpw,), jnp.int32),
            pltpu.VMEM((nchunk, 64), jnp.int32),
            pltpu.VMEM((bpw, 2 * EMB), jnp.float32),
            pltpu.VMEM((N_HOUR, 2 * EMB), jnp.float32),
            pltpu.VMEM_SHARED((N_MONTH * N_HOUR, 2 * EMB), jnp.float32),
            pltpu.SemaphoreType.DMA,
            pltpu.SemaphoreType.DMA,
            pltpu.SemaphoreType.DMA,
            pltpu.SemaphoreType.DMA,
        ],
    )
    def fused(mon_hbm, hr_hbm, months_hbm, hours_hbm, out_hbm,
              mon_v, hr_v, idx_v, rows_v, buf_v, tbl2_s,
              sem_i, sem_b, sem_g, sem_w):
        cid = lax.axis_index("c")
        sid = lax.axis_index("s")
        wid = sid * nc + cid
        base = wid * bpw

        ld_m = pltpu.async_copy(months_hbm.at[pl.ds(base, bpw)], mon_v, sem_i)
        ld_h = pltpu.async_copy(hours_hbm.at[pl.ds(base, bpw)], hr_v, sem_i)

        @pl.when(sid < N_MONTH)
        def _():
            stage = [
                pltpu.async_copy(mon_hbm.at[sid],
                                 buf_v.at[0, pl.ds(0, EMB)], sem_b)
            ] + [
                pltpu.async_copy(hr_hbm.at[h],
                                 buf_v.at[h, pl.ds(EMB, EMB)], sem_b)
                for h in range(N_HOUR)
            ]
            for c in stage:
                c.wait()
            for c4 in range(EMB // 16):
                v0 = buf_v[0, pl.ds(c4 * 16, 16)]

                def rep(h, carry):
                    buf_v[h, pl.ds(c4 * 16, 16)] = carry
                    return carry

                lax.fori_loop(1, N_HOUR, rep, v0)
            pltpu.async_copy(
                buf_v, tbl2_s.at[pl.ds(sid * N_HOUR, N_HOUR)], sem_b
            ).wait()

        ld_m.wait()
        ld_h.wait()

        def body(j, carry):
            m = mon_v[pl.ds(j * 16, 16)]
            h = hr_v[pl.ds(j * 16, 16)]
            r = lax.shift_right_logical(j, 2)
            cb = (j & 3) * 16
            idx_v[r, pl.ds(cb, 16)] = m * N_HOUR + h
            return carry

        lax.fori_loop(0, bpw // 16, body, 0)

        plsc.subcore_barrier()

        gathers = [
            pltpu.async_copy(tbl2_s.at[idx_v.at[k]],
                             rows_v.at[pl.ds(k * 64, 64)], sem_g)
            for k in range(nchunk)
        ]
        writes = []
        for k in range(nchunk):
            gathers[k].wait()
            writes.append(
                pltpu.async_copy(rows_v.at[pl.ds(k * 64, 64)],
                                 out_hbm.at[pl.ds(base + k * 64, 64)],
                                 sem_w))
        for w in writes:
            w.wait()

    return fused


def kernel(months, hours, month_table, hour_table):
    return _build_call()(month_table, hour_table,
                         months.astype(jnp.int32),
                         hours.astype(jnp.int32))


# speedup vs baseline: 1.0340x; 1.0117x over previous
import functools

import jax
import jax.numpy as jnp
from jax import lax
from jax.experimental import pallas as pl
from jax.experimental.pallas import tpu as pltpu
from jax.experimental.pallas import tpu_sc as plsc

EMB = 64
BATCH = 16384
N_MONTH = 12
N_HOUR = 24


def _build_call():
    info = plsc.get_sparse_core_info()
    nc, ns = info.num_cores, info.num_subcores
    nw = nc * ns
    bpw = BATCH // nw
    nchunk = bpw // 64

    mesh = plsc.VectorSubcoreMesh(core_axis_name="c", subcore_axis_name="s")

    @functools.partial(
        pl.kernel,
        mesh=mesh,
        out_type=jax.ShapeDtypeStruct((BATCH, 2 * EMB), jnp.float32),
        scratch_types=[
            pltpu.VMEM((bpw,), jnp.int32),
            pltpu.VMEM((bpw,), jnp.int32),
            pltpu.VMEM((nchunk, 64), jnp.int32),
            pltpu.VMEM((bpw, 2 * EMB), jnp.float32),
            pltpu.VMEM((N_HOUR, 2 * EMB), jnp.float32),
            pltpu.VMEM_SHARED((N_MONTH * N_HOUR, 2 * EMB), jnp.float32),
            pltpu.SemaphoreType.DMA,
            pltpu.SemaphoreType.DMA,
            pltpu.SemaphoreType.DMA,
            pltpu.SemaphoreType.DMA,
        ],
    )
    def fused(mon_hbm, hr_hbm, months_hbm, hours_hbm, out_hbm,
              mon_v, hr_v, idx_v, rows_v, buf_v, tbl2_s,
              sem_i, sem_b, sem_g, sem_w):
        cid = lax.axis_index("c")
        sid = lax.axis_index("s")
        wid = sid * nc + cid
        base = wid * bpw

        ld_m = pltpu.async_copy(months_hbm.at[pl.ds(base, bpw)], mon_v, sem_i)
        ld_h = pltpu.async_copy(hours_hbm.at[pl.ds(base, bpw)], hr_v, sem_i)

        @pl.when(sid < N_MONTH)
        def _():
            for h in range(N_HOUR):
                pltpu.async_copy(hr_hbm.at[h],
                                 buf_v.at[h, pl.ds(EMB, EMB)], sem_b)
            pltpu.async_copy(mon_hbm.at[sid],
                             buf_v.at[0, pl.ds(0, EMB)], sem_b)

        ld_m.wait()
        ld_h.wait()

        def body(j, carry):
            m = mon_v[pl.ds(j * 16, 16)]
            h = hr_v[pl.ds(j * 16, 16)]
            r = lax.shift_right_logical(j, 2)
            cb = (j & 3) * 16
            idx_v[r, pl.ds(cb, 16)] = m * N_HOUR + h
            return carry

        lax.fori_loop(0, bpw // 16, body, 0)

        @pl.when(sid < N_MONTH)
        def _():
            for h in range(N_HOUR):
                pltpu.make_async_copy(hr_hbm.at[h],
                                      buf_v.at[h, pl.ds(EMB, EMB)],
                                      sem_b).wait()
            pltpu.make_async_copy(mon_hbm.at[sid],
                                  buf_v.at[0, pl.ds(0, EMB)], sem_b).wait()
            for c4 in range(EMB // 16):
                v0 = buf_v[0, pl.ds(c4 * 16, 16)]

                def rep(h, carry):
                    buf_v[h, pl.ds(c4 * 16, 16)] = carry
                    return carry

                lax.fori_loop(1, N_HOUR, rep, v0)
            pltpu.async_copy(
                buf_v, tbl2_s.at[pl.ds(sid * N_HOUR, N_HOUR)], sem_b
            ).wait()

        plsc.subcore_barrier()

        gathers = [
            pltpu.async_copy(tbl2_s.at[idx_v.at[k]],
                             rows_v.at[pl.ds(k * 64, 64)], sem_g)
            for k in range(nchunk)
        ]
        writes = []
        for k in range(nchunk):
            gathers[k].wait()
            writes.append(
                pltpu.async_copy(rows_v.at[pl.ds(k * 64, 64)],
                                 out_hbm.at[pl.ds(base + k * 64, 64)],
                                 sem_w))
        for w in writes:
            w.wait()

    return fused


def kernel(months, hours, month_table, hour_table):
    return _build_call()(month_table, hour_table,
                         months.astype(jnp.int32),
                         hours.astype(jnp.int32))
